# baseline (speedup 1.0000x reference)
import functools

import jax
import jax.numpy as jnp
from jax import lax
from jax.experimental import pallas as pl
from jax.experimental.pallas import tpu as pltpu
from jax.experimental.pallas import tpu_sc as plsc

N = 10000
E = 320000
D = 128

NC = 2
NS = 16
NW = NC * NS

CH = 128
K = 80
KA = 160
KB = 0
SROWS = 48
NROW = NS * (KA + KB) + 72
EPAD = NROW * CH
NPAD = 10112
RPT = NPAD // NS

_mesh = plsc.VectorSubcoreMesh(
    core_axis_name="c", subcore_axis_name="s", num_cores=NC, num_subcores=NS
)


HN = 10240


def _deg_body(dst_hbm, zeros_hbm, out_hbm, idx_v, hist):
    c = lax.axis_index("c")
    s = lax.axis_index("s")
    w = c * NS + s
    pltpu.sync_copy(zeros_hbm, hist)
    pltpu.sync_copy(dst_hbm.at[pl.ds(w * K, K)], idx_v)
    ones = jnp.full((16,), 1.0, jnp.float32)

    def body(j, carry):
        for l in range(CH // 16):
            idx = idx_v[j, pl.ds(l * 16, 16)]
            plsc.addupdate_scatter(hist, [idx], ones)
        return carry

    lax.fori_loop(0, K, body, 0)
    pltpu.sync_copy(hist, out_hbm.at[w])


_deg_kernel = pl.kernel(
    _deg_body,
    out_type=jax.ShapeDtypeStruct((NW, HN), jnp.float32),
    mesh=_mesh,
    scratch_types=[
        pltpu.VMEM((K, CH), jnp.int32),
        pltpu.VMEM((HN,), jnp.float32),
    ],
    compiler_params=pltpu.CompilerParams(needs_layout_passes=False),
)


def _agg_body(table_hbm, src_hbm, dst_hbm, zeros_hbm, out_hbm,
              src_v, dst_v, rows_a, rows_b, agg_sp, sem_a, sem_b):
    c = lax.axis_index("c")
    s = lax.axis_index("s")
    base = jnp.where(c == 0, s * KA, NS * KA + s * KB)
    nk = jnp.where(c == 0, KA, KB)
    nstages = (nk + SROWS - 1) // SROWS
    pltpu.sync_copy(zeros_hbm.at[pl.ds(s * RPT, RPT)], agg_sp.at[pl.ds(s * RPT, RPT)])
    plsc.subcore_barrier()

    def stage(st, carry):
        njs = jnp.minimum(SROWS, nk - st * SROWS)
        pltpu.sync_copy(src_hbm.at[pl.ds(base + st * SROWS, SROWS)], src_v)
        pltpu.sync_copy(dst_hbm.at[pl.ds(base + st * SROWS, SROWS)], dst_v)
        pltpu.async_copy(table_hbm.at[src_v.at[0]], rows_a, sem_a)

        def body(t, inner):
            j0 = 2 * t
            j1 = j0 + 1
            pltpu.make_async_copy(table_hbm.at[src_v.at[0]], rows_a, sem_a).wait()
            pltpu.async_copy(table_hbm.at[src_v.at[j1]], rows_b, sem_b)
            pltpu.sync_copy(rows_a, agg_sp.at[dst_v.at[j0]], add=True)
            pltpu.make_async_copy(table_hbm.at[src_v.at[0]], rows_b, sem_b).wait()
            pltpu.async_copy(
                table_hbm.at[src_v.at[jnp.minimum(j0 + 2, njs - 2)]], rows_a, sem_a)
            pltpu.sync_copy(rows_b, agg_sp.at[dst_v.at[j1]], add=True)
            return inner

        lax.fori_loop(0, njs // 2, body, 0)
        pltpu.make_async_copy(table_hbm.at[src_v.at[0]], rows_a, sem_a).wait()
        return carry

    lax.fori_loop(0, nstages, stage, 0)
    plsc.subcore_barrier()
    pltpu.sync_copy(agg_sp.at[pl.ds(s * RPT, RPT)], out_hbm.at[c, pl.ds(s * RPT, RPT)])


_agg_kernel = pl.kernel(
    _agg_body,
    out_type=jax.ShapeDtypeStruct((NC, NPAD, D), jnp.float32),
    mesh=_mesh,
    scratch_types=[
        pltpu.VMEM((SROWS, CH), jnp.int32),
        pltpu.VMEM((SROWS, CH), jnp.int32),
        pltpu.VMEM((CH, D), jnp.float32),
        pltpu.VMEM((CH, D), jnp.float32),
        pltpu.VMEM_SHARED((NPAD, D), jnp.float32),
        pltpu.SemaphoreType.DMA,
        pltpu.SemaphoreType.DMA,
    ],
)

R = 1000


def _tc_b_body(x_ref, w_ref, degp_ref, scaled_ref, dinvb_ref):
    deg = jnp.sum(degp_ref[...], axis=0) + 1.0
    dinv1 = lax.rsqrt(deg)
    dinv = jnp.broadcast_to(dinv1, (R, D))
    xw = jnp.dot(x_ref[...], w_ref[...], preferred_element_type=jnp.float32)
    scaled_ref[...] = xw * dinv
    dinvb_ref[...] = dinv


def _tc_d_body(aggp_ref, scaled_ref, dinvb_ref, w_ref, b_ref, out_ref):
    agg = aggp_ref[0] + aggp_ref[1] + scaled_ref[...]
    h = jnp.maximum(dinvb_ref[...] * agg + b_ref[...], 0.0)
    out_ref[...] = (
        jnp.dot(h, w_ref[...], preferred_element_type=jnp.float32)
        * dinvb_ref[...]
    )


def _tc_f_body(aggp_ref, scaled_ref, dinvb_ref, b_ref, out_ref):
    agg = aggp_ref[0] + aggp_ref[1] + scaled_ref[...]
    out_ref[...] = dinvb_ref[...] * agg + b_ref[...]


_row_spec = pl.BlockSpec((R, D), lambda i: (i, 0))
_w_spec = pl.BlockSpec((D, D), lambda i: (0, 0))
_b_spec = pl.BlockSpec((1, D), lambda i: (0, 0))
_degp_spec = pl.BlockSpec((NW, R, 1), lambda i: (0, i, 0))
_aggp_spec = pl.BlockSpec((NC, R, D), lambda i: (0, i, 0))

_tc_b = pl.pallas_call(
    _tc_b_body,
    grid=(N // R,),
    in_specs=[_row_spec, _w_spec, _degp_spec],
    out_specs=[_row_spec, _row_spec],
    out_shape=[
        jax.ShapeDtypeStruct((N, D), jnp.float32),
        jax.ShapeDtypeStruct((N, D), jnp.float32),
    ],
)

_tc_d = pl.pallas_call(
    _tc_d_body,
    grid=(N // R,),
    in_specs=[_aggp_spec, _row_spec, _row_spec, _w_spec, _b_spec],
    out_specs=_row_spec,
    out_shape=jax.ShapeDtypeStruct((N, D), jnp.float32),
)

_tc_f = pl.pallas_call(
    _tc_f_body,
    grid=(N // R,),
    in_specs=[_aggp_spec, _row_spec, _row_spec, _b_spec],
    out_specs=_row_spec,
    out_shape=jax.ShapeDtypeStruct((N, D), jnp.float32),
)


@jax.jit
def kernel(x, edge_index, W1, b1, W2, b2):
    src = edge_index[0].astype(jnp.int32)
    dst = edge_index[1].astype(jnp.int32)
    src_p = jnp.concatenate(
        [src, jnp.zeros((EPAD - E,), jnp.int32)]).reshape(NROW, CH)
    pad_dst = N + jnp.arange(EPAD - E, dtype=jnp.int32) % (NPAD - N)
    dst_p = jnp.concatenate([dst, pad_dst]).reshape(NROW, CH)

    zeros_deg = jnp.zeros((HN,), jnp.float32)
    zeros_big = jnp.zeros((NPAD, D), jnp.float32)

    degp = _deg_kernel(dst_p, zeros_deg).reshape(NW, HN, 1)

    scaled1, dinvb = _tc_b(x, W1, degp)
    agg1 = _agg_kernel(scaled1, src_p, dst_p, zeros_big)
    scaled2 = _tc_d(agg1, scaled1, dinvb, W2, b1.reshape(1, D))
    agg2 = _agg_kernel(scaled2, src_p, dst_p, zeros_big)
    out = _tc_f(agg2, scaled2, dinvb, b2.reshape(1, D))
    return out


# speedup vs baseline: 1.3953x; 1.3953x over previous
import functools

import jax
import jax.numpy as jnp
from jax import lax
from jax.experimental import pallas as pl
from jax.experimental.pallas import tpu as pltpu
from jax.experimental.pallas import tpu_sc as plsc

N = 10000
E = 320000
D = 128

NC = 2
NS = 16
NW = NC * NS

CH = 128
K = 80
KA = 136
KB = 24
SROWS = 48
NROW = NS * (KA + KB) + 72
EPAD = NROW * CH
NPAD = 10112
RPT = NPAD // NS

_mesh = plsc.VectorSubcoreMesh(
    core_axis_name="c", subcore_axis_name="s", num_cores=NC, num_subcores=NS
)


HN = 10240


def _deg_body(dst_hbm, zeros_hbm, out_hbm, idx_v, hist):
    c = lax.axis_index("c")
    s = lax.axis_index("s")
    w = c * NS + s
    pltpu.sync_copy(zeros_hbm, hist)
    pltpu.sync_copy(dst_hbm.at[pl.ds(w * K, K)], idx_v)
    ones = jnp.full((16,), 1.0, jnp.float32)

    def body(j, carry):
        for l in range(CH // 16):
            idx = idx_v[j, pl.ds(l * 16, 16)]
            plsc.addupdate_scatter(hist, [idx], ones)
        return carry

    lax.fori_loop(0, K, body, 0)
    pltpu.sync_copy(hist, out_hbm.at[w])


_deg_kernel = pl.kernel(
    _deg_body,
    out_type=jax.ShapeDtypeStruct((NW, HN), jnp.float32),
    mesh=_mesh,
    scratch_types=[
        pltpu.VMEM((K, CH), jnp.int32),
        pltpu.VMEM((HN,), jnp.float32),
    ],
    compiler_params=pltpu.CompilerParams(needs_layout_passes=False),
)


def _agg_body(table_hbm, src_hbm, dst_hbm, zeros_hbm, out_hbm,
              src_v, dst_v, rows_a, rows_b, agg_sp, sem_a, sem_b):
    c = lax.axis_index("c")
    s = lax.axis_index("s")
    base = jnp.where(c == 0, s * KA, NS * KA + s * KB)
    nk = jnp.where(c == 0, KA, KB)
    nstages = (nk + SROWS - 1) // SROWS
    pltpu.sync_copy(zeros_hbm.at[pl.ds(s * RPT, RPT)], agg_sp.at[pl.ds(s * RPT, RPT)])
    plsc.subcore_barrier()

    def stage(st, carry):
        njs = jnp.minimum(SROWS, nk - st * SROWS)
        pltpu.sync_copy(src_hbm.at[pl.ds(base + st * SROWS, SROWS)], src_v)
        pltpu.sync_copy(dst_hbm.at[pl.ds(base + st * SROWS, SROWS)], dst_v)
        pltpu.async_copy(table_hbm.at[src_v.at[0]], rows_a, sem_a)

        def body(t, inner):
            j0 = 2 * t
            j1 = j0 + 1
            pltpu.make_async_copy(table_hbm.at[src_v.at[0]], rows_a, sem_a).wait()
            pltpu.async_copy(table_hbm.at[src_v.at[j1]], rows_b, sem_b)
            pltpu.sync_copy(rows_a, agg_sp.at[dst_v.at[j0]], add=True)
            pltpu.make_async_copy(table_hbm.at[src_v.at[0]], rows_b, sem_b).wait()
            pltpu.async_copy(
                table_hbm.at[src_v.at[jnp.minimum(j0 + 2, njs - 2)]], rows_a, sem_a)
            pltpu.sync_copy(rows_b, agg_sp.at[dst_v.at[j1]], add=True)
            return inner

        lax.fori_loop(0, njs // 2, body, 0)
        pltpu.make_async_copy(table_hbm.at[src_v.at[0]], rows_a, sem_a).wait()
        return carry

    lax.fori_loop(0, nstages, stage, 0)
    plsc.subcore_barrier()
    pltpu.sync_copy(agg_sp.at[pl.ds(s * RPT, RPT)], out_hbm.at[c, pl.ds(s * RPT, RPT)])


_agg_kernel = pl.kernel(
    _agg_body,
    out_type=jax.ShapeDtypeStruct((NC, NPAD, D), jnp.float32),
    mesh=_mesh,
    scratch_types=[
        pltpu.VMEM((SROWS, CH), jnp.int32),
        pltpu.VMEM((SROWS, CH), jnp.int32),
        pltpu.VMEM((CH, D), jnp.float32),
        pltpu.VMEM((CH, D), jnp.float32),
        pltpu.VMEM_SHARED((NPAD, D), jnp.float32),
        pltpu.SemaphoreType.DMA,
        pltpu.SemaphoreType.DMA,
    ],
)

R = 1000
RB = 1024


def _tc_b_body(x_ref, w_ref, degp_ref, scaled_ref, dinvb_ref):
    deg = jnp.sum(degp_ref[...], axis=0) + 1.0
    dinv_lane = lax.rsqrt(deg)
    dinv = jnp.broadcast_to(dinv_lane[:, :, None], (8, 128, D)).reshape(RB, D)
    xw = jnp.dot(x_ref[...], w_ref[...], preferred_element_type=jnp.float32)
    scaled_ref[...] = xw * dinv
    dinvb_ref[...] = dinv


def _tc_d_body(aggp_ref, scaled_ref, dinvb_ref, w_ref, b_ref, out_ref):
    agg = aggp_ref[0] + aggp_ref[1] + scaled_ref[...]
    h = jnp.maximum(dinvb_ref[...] * agg + b_ref[...], 0.0)
    out_ref[...] = (
        jnp.dot(h, w_ref[...], preferred_element_type=jnp.float32)
        * dinvb_ref[...]
    )


def _tc_f_body(aggp_ref, scaled_ref, dinvb_ref, b_ref, out_ref):
    agg = aggp_ref[0] + aggp_ref[1] + scaled_ref[...]
    out_ref[...] = dinvb_ref[...] * agg + b_ref[...]


_row_spec = pl.BlockSpec((R, D), lambda i: (i, 0))
_w_spec = pl.BlockSpec((D, D), lambda i: (0, 0))
_b_spec = pl.BlockSpec((1, D), lambda i: (0, 0))
_degp_spec = pl.BlockSpec((NW, 8, 128), lambda i: (0, i, 0))
_rowb_spec = pl.BlockSpec((RB, D), lambda i: (i, 0))
_aggp_spec = pl.BlockSpec((NC, R, D), lambda i: (0, i, 0))

_tc_b = pl.pallas_call(
    _tc_b_body,
    grid=((N + RB - 1) // RB,),
    in_specs=[_rowb_spec, pl.BlockSpec((D, D), lambda i: (0, 0)), _degp_spec],
    out_specs=[_rowb_spec, _rowb_spec],
    out_shape=[
        jax.ShapeDtypeStruct((N, D), jnp.float32),
        jax.ShapeDtypeStruct((N, D), jnp.float32),
    ],
)

_tc_d = pl.pallas_call(
    _tc_d_body,
    grid=(N // R,),
    in_specs=[_aggp_spec, _row_spec, _row_spec, _w_spec, _b_spec],
    out_specs=_row_spec,
    out_shape=jax.ShapeDtypeStruct((N, D), jnp.float32),
)

_tc_f = pl.pallas_call(
    _tc_f_body,
    grid=(N // R,),
    in_specs=[_aggp_spec, _row_spec, _row_spec, _b_spec],
    out_specs=_row_spec,
    out_shape=jax.ShapeDtypeStruct((N, D), jnp.float32),
)


@jax.jit
def kernel(x, edge_index, W1, b1, W2, b2):
    src = edge_index[0].astype(jnp.int32)
    dst = edge_index[1].astype(jnp.int32)
    src_p = jnp.concatenate(
        [src, jnp.zeros((EPAD - E,), jnp.int32)]).reshape(NROW, CH)
    pad_dst = N + jnp.arange(EPAD - E, dtype=jnp.int32) % (NPAD - N)
    dst_p = jnp.concatenate([dst, pad_dst]).reshape(NROW, CH)

    zeros_deg = jnp.zeros((HN,), jnp.float32)
    zeros_big = jnp.zeros((NPAD, D), jnp.float32)

    degp = _deg_kernel(dst_p, zeros_deg).reshape(NW, HN // 128, 128)

    scaled1, dinvb = _tc_b(x, W1, degp)
    agg1 = _agg_kernel(scaled1, src_p, dst_p, zeros_big)
    scaled2 = _tc_d(agg1, scaled1, dinvb, W2, b1.reshape(1, D))
    agg2 = _agg_kernel(scaled2, src_p, dst_p, zeros_big)
    out = _tc_f(agg2, scaled2, dinvb, b2.reshape(1, D))
    return out
